# baseline (speedup 1.0000x reference)
import functools

import jax
import jax.numpy as jnp
from jax import lax
from jax.experimental import pallas as pl
from jax.experimental.pallas import tpu as pltpu
from jax.experimental.pallas import tpu_sc as plsc

NUM_EMB = 1_000_000
DIM = 64
PAD_DIM = 128
BATCH = 16384
FIELDS = 26

NC = 2
NS = 16
NW = NC * NS
B_PER_W = BATCH // NW
BLK = 128
NBLK = B_PER_W // BLK

_mesh = plsc.VectorSubcoreMesh(core_axis_name="c", subcore_axis_name="s")


@functools.partial(
    pl.kernel,
    out_type=jax.ShapeDtypeStruct((FIELDS, DIM, BATCH), jnp.float32),
    mesh=_mesh,
    scratch_types=[
        pltpu.VMEM((FIELDS, B_PER_W), jnp.int32),
        pltpu.VMEM((BLK, PAD_DIM), jnp.float32),
        pltpu.VMEM((DIM, BLK), jnp.float32),
        pltpu.SemaphoreType.DMA,
    ],
    compiler_params=pltpu.CompilerParams(needs_layout_passes=False),
)
def _gather_kernel(idxt_hbm, table_hbm, out_hbm, idxt_v, rows_v, trans_v, sem):
    wid = lax.axis_index("s") * NC + lax.axis_index("c")
    base = wid * B_PER_W
    pltpu.sync_copy(idxt_hbm.at[:, pl.ds(base, B_PER_W)], idxt_v)
    lanes = lax.iota(jnp.int32, 16)

    def body(g, carry):
        f = g // NBLK
        bb = (g % NBLK) * BLK
        blk_idx = idxt_v.at[f, pl.ds(bb, BLK)]
        pltpu.async_copy(table_hbm.at[blk_idx], rows_v, sem).wait()
        for j in range(DIM):
            j_vec = jnp.full((16,), j, jnp.int32)
            for gg in range(BLK // 16):
                b_vec = lanes + (16 * gg)
                vals = plsc.load_gather(rows_v, [b_vec, j_vec])
                trans_v[j, pl.ds(16 * gg, 16)] = vals
        pltpu.sync_copy(trans_v, out_hbm.at[f, :, pl.ds(base + bb, BLK)])
        return carry

    lax.fori_loop(0, FIELDS * NBLK, body, 0)


def kernel(token_ids, weights):
    wpad = jnp.pad(weights, ((0, 0), (0, PAD_DIM - DIM)))
    outt = _gather_kernel(token_ids.astype(jnp.int32).T, wpad)
    return outt.transpose(2, 0, 1)


# speedup vs baseline: 1.5214x; 1.5214x over previous
import functools

import jax
import jax.numpy as jnp
from jax import lax
from jax.experimental import pallas as pl
from jax.experimental.pallas import tpu as pltpu
from jax.experimental.pallas import tpu_sc as plsc

NUM_EMB = 1_000_000
DIM = 64
PAD_DIM = 128
BATCH = 16384
FIELDS = 26

NC = 2
NS = 16
NW = NC * NS
B_PER_W = BATCH // NW
CHUNK_B = 16
NCHUNK = B_PER_W // CHUNK_B
NPAIR = NCHUNK // 2

_mesh = plsc.VectorSubcoreMesh(core_axis_name="c", subcore_axis_name="s")


@functools.partial(
    pl.kernel,
    out_type=jax.ShapeDtypeStruct((BATCH, FIELDS, DIM), jnp.float32),
    mesh=_mesh,
    scratch_types=[
        pltpu.VMEM((B_PER_W, FIELDS), jnp.int32),
        pltpu.VMEM((2, CHUNK_B, FIELDS, DIM), jnp.float32),
        pltpu.SemaphoreType.DMA,
        pltpu.SemaphoreType.DMA,
    ],
    compiler_params=pltpu.CompilerParams(
        use_tc_tiling_on_sc=False, needs_layout_passes=False
    ),
)
def _gather_kernel(idx_hbm, table_hbm, out_hbm, idx_v, rows_v, gsem0, gsem1):
    wid = lax.axis_index("s") * NC + lax.axis_index("c")
    base = wid * B_PER_W
    pltpu.sync_copy(idx_hbm.at[pl.ds(base, B_PER_W), :], idx_v)

    def fire(c, buf, sem):
        s = c * CHUNK_B
        for i in range(CHUNK_B):
            pltpu.async_copy(
                table_hbm.at[idx_v.at[s + i, :]], rows_v.at[buf, i], sem
            )

    def drain(buf, sem):
        for i in range(CHUNK_B):
            pltpu.make_async_copy(
                table_hbm.at[idx_v.at[i, :]], rows_v.at[buf, i], sem
            ).wait()

    def store(c, buf):
        pltpu.sync_copy(
            rows_v.at[buf], out_hbm.at[pl.ds(base + c * CHUNK_B, CHUNK_B)]
        )

    fire(0, 0, gsem0)

    def body(k, carry):
        del carry
        e, o = 2 * k, 2 * k + 1
        fire(o, 1, gsem1)
        drain(0, gsem0)
        store(e, 0)

        @pl.when(k < NPAIR - 1)
        def _():
            fire(o + 1, 0, gsem0)

        drain(1, gsem1)
        store(o, 1)
        return 0

    lax.fori_loop(0, NPAIR, body, 0)


def kernel(token_ids, weights):
    wpad = jnp.pad(weights, ((0, 0), (0, PAD_DIM - DIM)))
    w2 = wpad.reshape(2 * NUM_EMB, DIM)
    idx2 = token_ids.astype(jnp.int32) * 2
    return _gather_kernel(idx2, w2)
